# baseline (speedup 1.0000x reference)
import functools

import jax
import jax.numpy as jnp
from jax.experimental import pallas as pl
from jax.experimental.pallas import tpu as pltpu

DIM_ = 1024
E_ = 16
H_ = 4096
K_ = 2
B_ = 8
HCSZ = 1024
NHC = H_ // HCSZ


def _router_body(x_ref, wr_ref, br_ref, logits_ref, probs_ref, tki_ref, tkp_ref):
    xb = x_ref[...]
    lg = jnp.dot(xb, wr_ref[...], preferred_element_type=jnp.float32) + br_ref[...]
    logits_ref[...] = lg
    m = jnp.max(lg, axis=-1, keepdims=True)
    ex = jnp.exp(lg - m)
    pr = ex / jnp.sum(ex, axis=-1, keepdims=True)
    probs_ref[...] = pr
    lane = jax.lax.broadcasted_iota(jnp.int32, (B_, E_), 1)
    m1 = jnp.max(pr, axis=-1, keepdims=True)
    i1 = jnp.min(jnp.where(pr == m1, lane, E_), axis=-1, keepdims=True)
    pm = jnp.where(lane == i1, -jnp.inf, pr)
    m2 = jnp.max(pm, axis=-1, keepdims=True)
    i2 = jnp.min(jnp.where(pm == m2, lane, E_), axis=-1, keepdims=True)
    k_lane = jax.lax.broadcasted_iota(jnp.int32, (B_, K_), 1)
    tki_ref[...] = jnp.where(k_lane == 0, i1, i2)
    tkp_ref[...] = jnp.where(k_lane == 0, m1, m2)


def _ffn_body(sched_ref, x_ref, w1_ref, b1_ref, w2_ref, b2_ref, crow_ref, out_ref):
    hc = pl.program_id(0)
    i = pl.program_id(1)
    nact = sched_ref[E_]

    @pl.when((hc == 0) & (i == 0))
    def _init():
        out_ref[...] = jnp.zeros_like(out_ref)

    @pl.when(i < nact)
    def _compute():
        xb = x_ref[...]
        h = jnp.dot(xb, w1_ref[0], preferred_element_type=jnp.float32)
        h = h + b1_ref[0]
        g = 0.5 * h * (1.0 + jax.lax.erf(h * 0.7071067811865476))
        p = jnp.dot(g, w2_ref[0], preferred_element_type=jnp.float32)
        p = p + jnp.where(hc == 0, 1.0, 0.0) * b2_ref[0]
        lane = jax.lax.broadcasted_iota(jnp.int32, (B_, E_), 1)
        col = jnp.sum(jnp.where(lane == i, crow_ref[...], 0.0), axis=1,
                      keepdims=True)
        out_ref[...] = out_ref[...] + col * p


@functools.partial(jax.jit, static_argnums=())
def kernel(x, Wr, br, W1, b1, W2, b2):
    xf = x.reshape(B_, DIM_)
    logits, probs, tki, tkp = pl.pallas_call(
        _router_body,
        out_shape=(
            jax.ShapeDtypeStruct((B_, E_), jnp.float32),
            jax.ShapeDtypeStruct((B_, E_), jnp.float32),
            jax.ShapeDtypeStruct((B_, K_), jnp.int32),
            jax.ShapeDtypeStruct((B_, K_), jnp.float32),
        ),
    )(xf, Wr, br.reshape(1, E_))

    flat_e = tki.reshape(-1)
    flat_p = tkp.reshape(-1)
    tok = jnp.repeat(jnp.arange(B_, dtype=jnp.int32), K_)
    comb = jnp.zeros((B_, E_), jnp.float32).at[tok, flat_e].add(flat_p)
    active = jnp.zeros((E_,), jnp.bool_).at[flat_e].set(True)
    pos = jnp.cumsum(active.astype(jnp.int32)) - 1
    nact = jnp.sum(active.astype(jnp.int32))
    sched = jnp.zeros((E_,), jnp.int32).at[
        jnp.where(active, pos, E_)
    ].set(jnp.arange(E_, dtype=jnp.int32), mode="drop")
    sched = jnp.where(jnp.arange(E_) < nact, sched, sched[nact - 1])
    crowT = jnp.where(jnp.arange(E_)[None, :] < nact, comb[:, sched], 0.0)
    sched_arg = jnp.concatenate([sched, nact[None]])

    grid_spec = pltpu.PrefetchScalarGridSpec(
        num_scalar_prefetch=1,
        grid=(NHC, E_),
        in_specs=[
            pl.BlockSpec((B_, DIM_), lambda hc, i, s: (0, 0)),
            pl.BlockSpec((1, DIM_, HCSZ), lambda hc, i, s: (s[i], 0, hc)),
            pl.BlockSpec((1, 1, HCSZ), lambda hc, i, s: (s[i], 0, hc)),
            pl.BlockSpec((1, HCSZ, DIM_), lambda hc, i, s: (s[i], hc, 0)),
            pl.BlockSpec((1, 1, DIM_), lambda hc, i, s: (s[i], 0, 0)),
            pl.BlockSpec((B_, E_), lambda hc, i, s: (0, 0)),
        ],
        out_specs=pl.BlockSpec((B_, DIM_), lambda hc, i, s: (0, 0)),
    )
    mixed = pl.pallas_call(
        _ffn_body,
        grid_spec=grid_spec,
        out_shape=jax.ShapeDtypeStruct((B_, DIM_), jnp.float32),
    )(sched_arg, xf, W1, b1.reshape(E_, 1, H_), W2, b2.reshape(E_, 1, DIM_), crowT)

    return (
        mixed.reshape(B_, 1, DIM_),
        logits.reshape(B_, 1, E_),
        probs.reshape(B_, 1, E_),
        tki.reshape(B_, 1, K_),
        tkp.reshape(B_, 1, K_),
    )


# speedup vs baseline: 1.1976x; 1.1976x over previous
import jax
import jax.numpy as jnp
from jax.experimental import pallas as pl
from jax.experimental.pallas import tpu as pltpu

DIM_ = 1024
E_ = 16
H_ = 4096
K_ = 2
B_ = 8
HCSZ = 2048
NHC = H_ // HCSZ


def _router_body(x_ref, wr_ref, br_ref, logits_ref, probs_ref, tki_ref,
                 tkp_ref, sched_ref, nact_ref, crow_ref):
    xb = x_ref[...]
    lg = jnp.dot(xb, wr_ref[...], preferred_element_type=jnp.float32) + br_ref[...]
    logits_ref[...] = lg
    m = jnp.max(lg, axis=-1, keepdims=True)
    ex = jnp.exp(lg - m)
    pr = ex / jnp.sum(ex, axis=-1, keepdims=True)
    probs_ref[...] = pr

    lane = jax.lax.broadcasted_iota(jnp.int32, (B_, E_), 1)
    m1 = jnp.max(pr, axis=-1, keepdims=True)
    i1 = jnp.min(jnp.where(pr == m1, lane, E_), axis=-1, keepdims=True)
    pm = jnp.where(lane == i1, -jnp.inf, pr)
    m2 = jnp.max(pm, axis=-1, keepdims=True)
    i2 = jnp.min(jnp.where(pm == m2, lane, E_), axis=-1, keepdims=True)
    k_lane = jax.lax.broadcasted_iota(jnp.int32, (B_, K_), 1)
    tki_ref[...] = jnp.where(k_lane == 0, i1, i2)
    tkp_ref[...] = jnp.where(k_lane == 0, m1, m2)

    comb = (jnp.where(lane == i1, m1, 0.0)
            + jnp.where(lane == i2, m2, 0.0))
    sel = jnp.where((lane == i1) | (lane == i2), 1.0, 0.0)
    active = jnp.max(sel, axis=0, keepdims=True)
    ecol = jax.lax.broadcasted_iota(jnp.int32, (E_, E_), 0)
    erow = jax.lax.broadcasted_iota(jnp.int32, (E_, E_), 1)
    strict_lt = jnp.where(ecol < erow, 1.0, 0.0)
    rank = jnp.dot(active, strict_lt, preferred_element_type=jnp.float32,
                   precision=jax.lax.Precision.HIGHEST)
    rank_i = (rank + 0.5).astype(jnp.int32)
    nact = jnp.sum(active, axis=1, keepdims=True)
    nact_i = nact.astype(jnp.int32)
    nact_ref[...] = nact_i
    g_mat = jnp.where((jnp.broadcast_to(rank_i, (E_, E_)) == ecol)
                      & (jnp.broadcast_to(active, (E_, E_)) > 0), 1.0, 0.0)
    sched_col = jnp.sum(g_mat * erow.astype(jnp.float32), axis=1,
                        keepdims=True)
    row_col = jax.lax.broadcasted_iota(jnp.int32, (E_, 1), 0)
    last = jnp.sum(jnp.where(row_col == nact_i - 1, sched_col, 0.0),
                   axis=0, keepdims=True)
    sched_ref[...] = (jnp.where(row_col < nact_i, sched_col,
                                jnp.broadcast_to(last, (E_, 1)))
                      + 0.5).astype(jnp.int32)
    crow_ref[...] = jax.lax.dot_general(
        comb, g_mat, dimension_numbers=(((1,), (1,)), ((), ())),
        preferred_element_type=jnp.float32,
        precision=jax.lax.Precision.HIGHEST)


def _ffn_body(sched_ref, nact_ref, x_ref, w1_ref, b1_ref, w2_ref, b2_ref,
              crow_ref, out_ref):
    hc = pl.program_id(0)
    i = pl.program_id(1)
    nact = nact_ref[0]

    @pl.when((hc == 0) & (i == 0))
    def _init():
        out_ref[...] = jnp.zeros_like(out_ref)

    @pl.when(i < nact)
    def _compute():
        xb = x_ref[...]
        h = jnp.dot(xb, w1_ref[0], preferred_element_type=jnp.float32)
        h = h + b1_ref[0]
        g = 0.5 * h * (1.0 + jax.lax.erf(h * 0.7071067811865476))
        p = jnp.dot(g, w2_ref[0], preferred_element_type=jnp.float32)
        p = p + jnp.where(hc == 0, 1.0, 0.0) * b2_ref[0]
        lane = jax.lax.broadcasted_iota(jnp.int32, (B_, E_), 1)
        col = jnp.sum(jnp.where(lane == i, crow_ref[...], 0.0), axis=1,
                      keepdims=True)
        out_ref[...] = out_ref[...] + col * p


def kernel(x, Wr, br, W1, b1, W2, b2):
    xf = x.reshape(B_, DIM_)
    logits, probs, tki, tkp, sched, nact, crowT = pl.pallas_call(
        _router_body,
        out_shape=(
            jax.ShapeDtypeStruct((B_, E_), jnp.float32),
            jax.ShapeDtypeStruct((B_, E_), jnp.float32),
            jax.ShapeDtypeStruct((B_, K_), jnp.int32),
            jax.ShapeDtypeStruct((B_, K_), jnp.float32),
            jax.ShapeDtypeStruct((E_, 1), jnp.int32),
            jax.ShapeDtypeStruct((1, 1), jnp.int32),
            jax.ShapeDtypeStruct((B_, E_), jnp.float32),
        ),
    )(xf, Wr, br.reshape(1, E_))

    grid_spec = pltpu.PrefetchScalarGridSpec(
        num_scalar_prefetch=2,
        grid=(NHC, E_),
        in_specs=[
            pl.BlockSpec((B_, DIM_), lambda hc, i, s, n: (0, 0)),
            pl.BlockSpec((1, DIM_, HCSZ), lambda hc, i, s, n: (s[i, 0], 0, hc)),
            pl.BlockSpec((1, 1, HCSZ), lambda hc, i, s, n: (s[i, 0], 0, hc)),
            pl.BlockSpec((1, HCSZ, DIM_), lambda hc, i, s, n: (s[i, 0], hc, 0)),
            pl.BlockSpec((1, 1, DIM_), lambda hc, i, s, n: (s[i, 0], 0, 0)),
            pl.BlockSpec((B_, E_), lambda hc, i, s, n: (0, 0)),
        ],
        out_specs=pl.BlockSpec((B_, DIM_), lambda hc, i, s, n: (0, 0)),
    )
    mixed = pl.pallas_call(
        _ffn_body,
        grid_spec=grid_spec,
        out_shape=jax.ShapeDtypeStruct((B_, DIM_), jnp.float32),
        compiler_params=pltpu.CompilerParams(
            vmem_limit_bytes=100 * 1024 * 1024),
    )(sched.reshape(E_, 1), nact.reshape(1,), xf, W1,
      b1.reshape(E_, 1, H_), W2, b2.reshape(E_, 1, DIM_), crowT)

    return (
        mixed.reshape(B_, 1, DIM_),
        logits.reshape(B_, 1, E_),
        probs.reshape(B_, 1, E_),
        tki.reshape(B_, 1, K_),
        tkp.reshape(B_, 1, K_),
    )


# speedup vs baseline: 1.2226x; 1.0209x over previous
import jax
import jax.numpy as jnp
from jax.experimental import pallas as pl
from jax.experimental.pallas import tpu as pltpu

DIM_ = 1024
E_ = 16
H_ = 4096
K_ = 2
B_ = 8
HCSZ = 2048
NHC = H_ // HCSZ


def _router_body(x_ref, wr_ref, br_ref, logits_ref, probs_ref, tki_ref,
                 tkp_ref, sched_ref, nact_ref, crow_ref):
    xb = x_ref[...]
    lg = jnp.dot(xb, wr_ref[...], preferred_element_type=jnp.float32) + br_ref[...]
    logits_ref[...] = lg
    m = jnp.max(lg, axis=-1, keepdims=True)
    ex = jnp.exp(lg - m)
    pr = ex / jnp.sum(ex, axis=-1, keepdims=True)
    probs_ref[...] = pr

    lane = jax.lax.broadcasted_iota(jnp.int32, (B_, E_), 1)
    m1 = jnp.max(pr, axis=-1, keepdims=True)
    i1 = jnp.min(jnp.where(pr == m1, lane, E_), axis=-1, keepdims=True)
    pm = jnp.where(lane == i1, -jnp.inf, pr)
    m2 = jnp.max(pm, axis=-1, keepdims=True)
    i2 = jnp.min(jnp.where(pm == m2, lane, E_), axis=-1, keepdims=True)
    k_lane = jax.lax.broadcasted_iota(jnp.int32, (B_, K_), 1)
    tki_ref[...] = jnp.where(k_lane == 0, i1, i2)
    tkp_ref[...] = jnp.where(k_lane == 0, m1, m2)

    comb = (jnp.where(lane == i1, m1, 0.0)
            + jnp.where(lane == i2, m2, 0.0))
    sel = jnp.where((lane == i1) | (lane == i2), 1.0, 0.0)
    active = jnp.max(sel, axis=0, keepdims=True)
    ecol = jax.lax.broadcasted_iota(jnp.int32, (E_, E_), 0)
    erow = jax.lax.broadcasted_iota(jnp.int32, (E_, E_), 1)
    strict_lt = jnp.where(ecol < erow, 1.0, 0.0)
    rank = jnp.dot(active, strict_lt, preferred_element_type=jnp.float32,
                   precision=jax.lax.Precision.HIGHEST)
    rank_i = (rank + 0.5).astype(jnp.int32)
    nact = jnp.sum(active, axis=1, keepdims=True)
    nact_i = nact.astype(jnp.int32)
    nact_ref[...] = nact_i
    g_mat = jnp.where((jnp.broadcast_to(rank_i, (E_, E_)) == ecol)
                      & (jnp.broadcast_to(active, (E_, E_)) > 0), 1.0, 0.0)
    sched_col = jnp.sum(g_mat * erow.astype(jnp.float32), axis=1,
                        keepdims=True)
    row_col = jax.lax.broadcasted_iota(jnp.int32, (E_, 1), 0)
    last = jnp.sum(jnp.where(row_col == nact_i - 1, sched_col, 0.0),
                   axis=0, keepdims=True)
    sched_ref[...] = (jnp.where(row_col < nact_i, sched_col,
                                jnp.broadcast_to(last, (E_, 1)))
                      + 0.5).astype(jnp.int32)
    crow_ref[...] = jax.lax.dot_general(
        comb, g_mat, dimension_numbers=(((1,), (1,)), ((), ())),
        preferred_element_type=jnp.float32,
        precision=jax.lax.Precision.HIGHEST)


def _ffn_body(sched_ref, nact_ref, x_ref, w1a_ref, w1b_ref, b1_ref,
              w2a_ref, w2b_ref, b2_ref, crow_ref, out_ref):
    hc = pl.program_id(0)
    i = pl.program_id(1)
    nact = nact_ref[0]

    @pl.when((hc == 0) & (i == 0))
    def _init():
        out_ref[...] = jnp.zeros_like(out_ref)

    @pl.when(i < nact)
    def _compute():
        xb = x_ref[...]
        h = (jnp.dot(xb[:, :DIM_ // 2], w1a_ref[0],
                     preferred_element_type=jnp.float32)
             + jnp.dot(xb[:, DIM_ // 2:], w1b_ref[0],
                       preferred_element_type=jnp.float32))
        h = h + b1_ref[0]
        g = 0.5 * h * (1.0 + jax.lax.erf(h * 0.7071067811865476))
        p = (jnp.dot(g[:, :HCSZ // 2], w2a_ref[0],
                     preferred_element_type=jnp.float32)
             + jnp.dot(g[:, HCSZ // 2:], w2b_ref[0],
                       preferred_element_type=jnp.float32))
        p = p + jnp.where(hc == 0, 1.0, 0.0) * b2_ref[0]
        lane = jax.lax.broadcasted_iota(jnp.int32, (B_, E_), 1)
        col = jnp.sum(jnp.where(lane == i, crow_ref[...], 0.0), axis=1,
                      keepdims=True)
        out_ref[...] = out_ref[...] + col * p


def kernel(x, Wr, br, W1, b1, W2, b2):
    xf = x.reshape(B_, DIM_)
    logits, probs, tki, tkp, sched, nact, crowT = pl.pallas_call(
        _router_body,
        out_shape=(
            jax.ShapeDtypeStruct((B_, E_), jnp.float32),
            jax.ShapeDtypeStruct((B_, E_), jnp.float32),
            jax.ShapeDtypeStruct((B_, K_), jnp.int32),
            jax.ShapeDtypeStruct((B_, K_), jnp.float32),
            jax.ShapeDtypeStruct((E_, 1), jnp.int32),
            jax.ShapeDtypeStruct((1, 1), jnp.int32),
            jax.ShapeDtypeStruct((B_, E_), jnp.float32),
        ),
    )(xf, Wr, br.reshape(1, E_))

    grid_spec = pltpu.PrefetchScalarGridSpec(
        num_scalar_prefetch=2,
        grid=(NHC, E_),
        in_specs=[
            pl.BlockSpec((B_, DIM_), lambda hc, i, s, n: (0, 0)),
            pl.BlockSpec((1, DIM_ // 2, HCSZ),
                         lambda hc, i, s, n: (s[i, 0], 0, hc)),
            pl.BlockSpec((1, DIM_ // 2, HCSZ),
                         lambda hc, i, s, n: (s[i, 0], 1, hc)),
            pl.BlockSpec((1, 1, HCSZ), lambda hc, i, s, n: (s[i, 0], 0, hc)),
            pl.BlockSpec((1, HCSZ // 2, DIM_),
                         lambda hc, i, s, n: (s[i, 0], 2 * hc, 0)),
            pl.BlockSpec((1, HCSZ // 2, DIM_),
                         lambda hc, i, s, n: (s[i, 0], 2 * hc + 1, 0)),
            pl.BlockSpec((1, 1, DIM_), lambda hc, i, s, n: (s[i, 0], 0, 0)),
            pl.BlockSpec((B_, E_), lambda hc, i, s, n: (0, 0)),
        ],
        out_specs=pl.BlockSpec((B_, DIM_), lambda hc, i, s, n: (0, 0)),
    )
    mixed = pl.pallas_call(
        _ffn_body,
        grid_spec=grid_spec,
        out_shape=jax.ShapeDtypeStruct((B_, DIM_), jnp.float32),
        compiler_params=pltpu.CompilerParams(
            vmem_limit_bytes=100 * 1024 * 1024),
    )(sched.reshape(E_, 1), nact.reshape(1,), xf, W1, W1,
      b1.reshape(E_, 1, H_), W2, W2, b2.reshape(E_, 1, DIM_), crowT)

    return (
        mixed.reshape(B_, 1, DIM_),
        logits.reshape(B_, 1, E_),
        probs.reshape(B_, 1, E_),
        tki.reshape(B_, 1, K_),
        tkp.reshape(B_, 1, K_),
    )
